# baseline (speedup 1.0000x reference)
import functools

import jax
import jax.numpy as jnp
from jax import lax
from jax.experimental import pallas as pl
from jax.experimental.pallas import tpu as pltpu
from jax.experimental.pallas import tpu_sc as plsc

N = 10000
NPAD = 10240
E = 320000
D = 128
NC = 2
NS = 16
NW = NC * NS
EPW = E // NW
CHUNK = 80
NSTEP = EPW // CHUNK
ROWS_PT = NPAD // NS


def _sc_aggregate(feat, edges):
    mesh = plsc.VectorSubcoreMesh(core_axis_name="c", subcore_axis_name="s")

    @functools.partial(
        pl.kernel,
        mesh=mesh,
        out_type=[
            jax.ShapeDtypeStruct((NC, NPAD, D), jnp.float32),
            jax.ShapeDtypeStruct((NC * NPAD,), jnp.float32),
        ],
        scratch_types=[
            pltpu.VMEM((4, CHUNK), jnp.int32),
            pltpu.VMEM((4, CHUNK), jnp.int32),
            pltpu.VMEM((CHUNK, D), jnp.float32),
            pltpu.VMEM((CHUNK, D), jnp.float32),
            pltpu.VMEM((CHUNK, D), jnp.float32),
            pltpu.VMEM((CHUNK, D), jnp.float32),
            pltpu.VMEM((CHUNK,), jnp.float32),
            pltpu.VMEM((ROWS_PT,), jnp.float32),
            pltpu.VMEM_SHARED((NPAD, D), jnp.float32),
            pltpu.VMEM_SHARED((NPAD,), jnp.float32),
        ] + [pltpu.SemaphoreType.DMA] * 12,
    )
    def k(feat_hbm, edges_hbm, sum_out, deg_out,
          srcv, dstv, rowsA, rowsB, rowsC, rowsD, ones_v, zbd_v,
          acc_sh, deg_sh, *sems):
        c = lax.axis_index("c")
        s = lax.axis_index("s")
        wid = c * NS + s
        rows_s = (rowsA, rowsB, rowsC, rowsD)
        isem = sems[0:4]
        gsem = sems[4:8]
        ssem = sems[8:12]

        def zb_body(i, carry):
            r = i // (D // 16)
            col = (i % (D // 16)) * 16
            rowsA[r, pl.ds(col, 16)] = jnp.zeros((16,), jnp.float32)
            return carry
        lax.fori_loop(0, CHUNK * (D // 16), zb_body, 0)

        def zbd_body(i, carry):
            zbd_v[pl.ds(i * 16, 16)] = jnp.zeros((16,), jnp.float32)
            return carry
        lax.fori_loop(0, ROWS_PT // 16, zbd_body, 0)

        def ones_body(i, carry):
            ones_v[pl.ds(i * 16, 16)] = jnp.ones((16,), jnp.float32)
            return carry
        lax.fori_loop(0, CHUNK // 16, ones_body, 0)

        row0_ = pl.multiple_of(s * ROWS_PT, 8)

        def z_body(j, carry):
            r = pl.multiple_of(row0_ + j * CHUNK, 8)
            pltpu.sync_copy(rowsA, acc_sh.at[pl.ds(r, CHUNK), :])
            return carry
        lax.fori_loop(0, ROWS_PT // CHUNK, z_body, 0)
        pltpu.sync_copy(zbd_v, deg_sh.at[pl.ds(row0_, ROWS_PT)])
        plsc.subcore_barrier()

        ebase = wid * EPW

        def start_idx(t, j):
            b = ebase + t * CHUNK
            pltpu.async_copy(edges_hbm.at[pl.ds(b, CHUNK)], srcv.at[j], isem[j])
            pltpu.async_copy(edges_hbm.at[pl.ds(E + b, CHUNK)], dstv.at[j],
                             isem[j])

        def wait_idx(t, j):
            b = ebase + t * CHUNK
            pltpu.make_async_copy(edges_hbm.at[pl.ds(b, CHUNK)], srcv.at[j],
                                  isem[j]).wait()
            pltpu.make_async_copy(edges_hbm.at[pl.ds(E + b, CHUNK)],
                                  dstv.at[j], isem[j]).wait()

        def start_g(j):
            pltpu.async_copy(feat_hbm.at[srcv.at[j]], rows_s[j], gsem[j])

        def wait_g(j):
            pltpu.make_async_copy(feat_hbm.at[srcv.at[j]], rows_s[j],
                                  gsem[j]).wait()

        def start_scat(j):
            pltpu.async_copy(rows_s[j], acc_sh.at[dstv.at[j]], ssem[j],
                             add=True)
            pltpu.async_copy(ones_v, deg_sh.at[dstv.at[j]], ssem[j], add=True)

        def wait_scat(j):
            pltpu.make_async_copy(rows_s[j], acc_sh.at[dstv.at[j]],
                                  ssem[j]).wait()
            pltpu.make_async_copy(ones_v, deg_sh.at[dstv.at[j]],
                                  ssem[j]).wait()

        start_idx(0, 0)
        start_idx(1, 1)
        wait_idx(0, 0)
        start_g(0)

        def e_body(i, carry):
            for j in range(4):
                t = i * 4 + j

                @pl.when(t + 2 < NSTEP)
                def _():
                    @pl.when(t >= 2)
                    def _():
                        wait_scat((j + 2) % 4)
                    start_idx(t + 2, (j + 2) % 4)

                @pl.when(t + 1 < NSTEP)
                def _():
                    wait_idx(t + 1, (j + 1) % 4)
                    start_g((j + 1) % 4)
                wait_g(j)
                start_scat(j)
            return carry
        lax.fori_loop(0, NSTEP // 4, e_body, 0)
        wait_g(0)
        start_scat(0)
        wait_scat(1)
        wait_scat(2)
        wait_scat(3)
        wait_scat(0)
        plsc.subcore_barrier()

        def o_body(j, carry):
            r = pl.multiple_of(row0_ + j * 128, 8)
            pltpu.sync_copy(acc_sh.at[pl.ds(r, 128), :],
                            sum_out.at[c, pl.ds(r, 128), :])
            return carry
        lax.fori_loop(0, ROWS_PT // 128, o_body, 0)
        pltpu.sync_copy(deg_sh.at[pl.ds(row0_, ROWS_PT)],
                        deg_out.at[pl.ds(c * NPAD + row0_, ROWS_PT)])

    return k(feat, edges)


BLK = 256
_HP = lax.Precision.HIGHEST


def _tc_merge(feat, sum2, deg2, W_self, W_neigh, bias):
    def body(f_ref, s_ref, d_ref, ws_ref, wn_ref, b_ref, o_ref):
        ssum = s_ref[0] + s_ref[1]
        deg = d_ref[0] + d_ref[1]
        h = ssum * (1.0 / jnp.maximum(deg, 1.0))
        o_ref[...] = (
            jnp.dot(f_ref[...], ws_ref[...],
                    preferred_element_type=jnp.float32, precision=_HP)
            + jnp.dot(h, wn_ref[...],
                      preferred_element_type=jnp.float32, precision=_HP)
            + b_ref[...])

    return pl.pallas_call(
        body,
        grid=((N + BLK - 1) // BLK,),
        in_specs=[
            pl.BlockSpec((BLK, D), lambda i: (i, 0)),
            pl.BlockSpec((NC, BLK, D), lambda i: (0, i, 0)),
            pl.BlockSpec((NC, BLK, 1), lambda i: (0, i, 0)),
            pl.BlockSpec((D, D), lambda i: (0, 0)),
            pl.BlockSpec((D, D), lambda i: (0, 0)),
            pl.BlockSpec((1, D), lambda i: (0, 0)),
        ],
        out_specs=pl.BlockSpec((BLK, D), lambda i: (i, 0)),
        out_shape=jax.ShapeDtypeStruct((N, D), jnp.float32),
    )(feat, sum2, deg2, W_self, W_neigh, bias)


def kernel(feat, edge_index, W_self, b_self, W_neigh, b_neigh):
    sum2, deg_flat = _sc_aggregate(feat, edge_index.reshape(-1))
    deg2 = deg_flat.reshape(NC, NPAD, 1)
    bias = (b_self + b_neigh).reshape(1, D)
    return _tc_merge(feat, sum2, deg2, W_self, W_neigh, bias)


# speedup vs baseline: 1.0028x; 1.0028x over previous
import functools

import jax
import jax.numpy as jnp
from jax import lax
from jax.experimental import pallas as pl
from jax.experimental.pallas import tpu as pltpu
from jax.experimental.pallas import tpu_sc as plsc

N = 10000
NPAD = 10240
E = 320000
D = 128
NC = 2
NS = 16
NW = NC * NS
EPW = E // NW
CHUNK = 80
NSTEP = EPW // CHUNK
ROWS_PT = NPAD // NS


def _sc_aggregate(feat, edges):
    mesh = plsc.VectorSubcoreMesh(core_axis_name="c", subcore_axis_name="s")

    @functools.partial(
        pl.kernel,
        mesh=mesh,
        out_type=[
            jax.ShapeDtypeStruct((NC, NPAD, D), jnp.float32),
            jax.ShapeDtypeStruct((NC * NPAD,), jnp.float32),
        ],
        scratch_types=[
            pltpu.VMEM((4, CHUNK), jnp.int32),
            pltpu.VMEM((4, CHUNK), jnp.int32),
            pltpu.VMEM((CHUNK, D), jnp.float32),
            pltpu.VMEM((CHUNK, D), jnp.float32),
            pltpu.VMEM((CHUNK, D), jnp.float32),
            pltpu.VMEM((CHUNK, D), jnp.float32),
            pltpu.VMEM((CHUNK,), jnp.float32),
            pltpu.VMEM((ROWS_PT,), jnp.float32),
            pltpu.VMEM_SHARED((NPAD, D), jnp.float32),
            pltpu.VMEM_SHARED((NPAD,), jnp.float32),
        ] + [pltpu.SemaphoreType.DMA] * 12,
    )
    def k(feat_hbm, edges_hbm, sum_out, deg_out,
          srcv, dstv, rowsA, rowsB, rowsC, rowsD, ones_v, zbd_v,
          acc_sh, deg_sh, *sems):
        c = lax.axis_index("c")
        s = lax.axis_index("s")
        wid = c * NS + s
        rows_s = (rowsA, rowsB, rowsC, rowsD)
        isem = sems[0:4]
        gsem = sems[4:8]
        ssem = sems[8:12]

        def zb_body(i, carry):
            r = i // (D // 16)
            col = (i % (D // 16)) * 16
            rowsA[r, pl.ds(col, 16)] = jnp.zeros((16,), jnp.float32)
            return carry
        lax.fori_loop(0, CHUNK * (D // 16), zb_body, 0)

        def zbd_body(i, carry):
            zbd_v[pl.ds(i * 16, 16)] = jnp.zeros((16,), jnp.float32)
            return carry
        lax.fori_loop(0, ROWS_PT // 16, zbd_body, 0)

        def ones_body(i, carry):
            ones_v[pl.ds(i * 16, 16)] = jnp.ones((16,), jnp.float32)
            return carry
        lax.fori_loop(0, CHUNK // 16, ones_body, 0)

        row0_ = pl.multiple_of(s * ROWS_PT, 8)

        def z_body(j, carry):
            r = pl.multiple_of(row0_ + j * CHUNK, 8)
            pltpu.sync_copy(rowsA, acc_sh.at[pl.ds(r, CHUNK), :])
            return carry
        lax.fori_loop(0, ROWS_PT // CHUNK, z_body, 0)
        pltpu.sync_copy(zbd_v, deg_sh.at[pl.ds(row0_, ROWS_PT)])
        plsc.subcore_barrier()

        ebase = wid * EPW

        def start_idx(t, j):
            b = ebase + t * CHUNK
            pltpu.async_copy(edges_hbm.at[pl.ds(b, CHUNK)], srcv.at[j], isem[j])
            pltpu.async_copy(edges_hbm.at[pl.ds(E + b, CHUNK)], dstv.at[j],
                             isem[j])

        def wait_idx(t, j):
            b = ebase + t * CHUNK
            pltpu.make_async_copy(edges_hbm.at[pl.ds(b, CHUNK)], srcv.at[j],
                                  isem[j]).wait()
            pltpu.make_async_copy(edges_hbm.at[pl.ds(E + b, CHUNK)],
                                  dstv.at[j], isem[j]).wait()

        def start_g(j):
            pltpu.async_copy(feat_hbm.at[srcv.at[j]], rows_s[j], gsem[j])

        def wait_g(j):
            pltpu.make_async_copy(feat_hbm.at[srcv.at[j]], rows_s[j],
                                  gsem[j]).wait()

        def start_scat(j):
            pltpu.async_copy(rows_s[j], acc_sh.at[dstv.at[j]], ssem[j],
                             add=True)
            pltpu.async_copy(ones_v, deg_sh.at[dstv.at[j]], ssem[j], add=True)

        def wait_scat(j):
            pltpu.make_async_copy(rows_s[j], acc_sh.at[dstv.at[j]],
                                  ssem[j]).wait()
            pltpu.make_async_copy(ones_v, deg_sh.at[dstv.at[j]],
                                  ssem[j]).wait()

        start_idx(0, 0)
        start_idx(1, 1)
        wait_idx(0, 0)
        start_g(0)

        def e_body(i, carry):
            for j in range(4):
                t = i * 4 + j

                @pl.when(t + 2 < NSTEP)
                def _():
                    @pl.when(t >= 2)
                    def _():
                        wait_scat((j + 2) % 4)
                    start_idx(t + 2, (j + 2) % 4)

                @pl.when(t + 1 < NSTEP)
                def _():
                    wait_idx(t + 1, (j + 1) % 4)
                    start_g((j + 1) % 4)
                wait_g(j)
                start_scat(j)
            return carry
        lax.fori_loop(0, NSTEP // 4, e_body, 0)
        wait_g(0)
        start_scat(0)
        wait_scat(1)
        wait_scat(2)
        wait_scat(3)
        wait_scat(0)
        plsc.subcore_barrier()

        def o_body(j, carry):
            r = pl.multiple_of(row0_ + j * 128, 8)
            pltpu.sync_copy(acc_sh.at[pl.ds(r, 128), :],
                            sum_out.at[c, pl.ds(r, 128), :])
            return carry
        lax.fori_loop(0, ROWS_PT // 128, o_body, 0)
        pltpu.sync_copy(deg_sh.at[pl.ds(row0_, ROWS_PT)],
                        deg_out.at[pl.ds(c * NPAD + row0_, ROWS_PT)])

    return k(feat, edges)


BLK = 256
_HP = lax.Precision.HIGHEST


def _tc_self(feat, W_self, bias):
    def body(f_ref, ws_ref, b_ref, o_ref):
        o_ref[...] = jnp.dot(
            f_ref[...], ws_ref[...], preferred_element_type=jnp.float32,
            precision=_HP) + b_ref[...]

    return pl.pallas_call(
        body,
        grid=((N + BLK - 1) // BLK,),
        in_specs=[
            pl.BlockSpec((BLK, D), lambda i: (i, 0)),
            pl.BlockSpec((D, D), lambda i: (0, 0)),
            pl.BlockSpec((1, D), lambda i: (0, 0)),
        ],
        out_specs=pl.BlockSpec((BLK, D), lambda i: (i, 0)),
        out_shape=jax.ShapeDtypeStruct((N, D), jnp.float32),
    )(feat, W_self, bias)


def _tc_combine(selfp, sum2, deg2, W_neigh):
    def body(p_ref, s_ref, d_ref, wn_ref, o_ref):
        ssum = s_ref[0] + s_ref[1]
        deg = d_ref[0] + d_ref[1]
        h = ssum * (1.0 / jnp.maximum(deg, 1.0))
        o_ref[...] = p_ref[...] + jnp.dot(
            h, wn_ref[...], preferred_element_type=jnp.float32, precision=_HP)

    return pl.pallas_call(
        body,
        grid=((N + BLK - 1) // BLK,),
        in_specs=[
            pl.BlockSpec((BLK, D), lambda i: (i, 0)),
            pl.BlockSpec((NC, BLK, D), lambda i: (0, i, 0)),
            pl.BlockSpec((NC, BLK, 1), lambda i: (0, i, 0)),
            pl.BlockSpec((D, D), lambda i: (0, 0)),
        ],
        out_specs=pl.BlockSpec((BLK, D), lambda i: (i, 0)),
        out_shape=jax.ShapeDtypeStruct((N, D), jnp.float32),
    )(selfp, sum2, deg2, W_neigh)


def kernel(feat, edge_index, W_self, b_self, W_neigh, b_neigh):
    sum2, deg_flat = _sc_aggregate(feat, edge_index.reshape(-1))
    deg2 = deg_flat.reshape(NC, NPAD, 1)
    bias = (b_self + b_neigh).reshape(1, D)
    selfp = _tc_self(feat, W_self, bias)
    return _tc_combine(selfp, sum2, deg2, W_neigh)


# speedup vs baseline: 1.0394x; 1.0365x over previous
import functools

import jax
import jax.numpy as jnp
from jax import lax
from jax.experimental import pallas as pl
from jax.experimental.pallas import tpu as pltpu
from jax.experimental.pallas import tpu_sc as plsc

N = 10000
NPAD = 10240
E = 320000
D = 128
NC = 2
NS = 16
NW = NC * NS
EPW = E // NW
CHUNK = 80
NSTEP = EPW // CHUNK
ROWS_PT = NPAD // NS


def _sc_aggregate(feat, edges):
    mesh = plsc.VectorSubcoreMesh(core_axis_name="c", subcore_axis_name="s")

    @functools.partial(
        pl.kernel,
        mesh=mesh,
        out_type=[
            jax.ShapeDtypeStruct((NC, NPAD, D), jnp.float32),
            jax.ShapeDtypeStruct((NC * NPAD,), jnp.float32),
        ],
        scratch_types=[
            pltpu.VMEM((4, CHUNK), jnp.int32),
            pltpu.VMEM((4, CHUNK), jnp.int32),
            pltpu.VMEM((CHUNK, D), jnp.float32),
            pltpu.VMEM((CHUNK, D), jnp.float32),
            pltpu.VMEM((CHUNK, D), jnp.float32),
            pltpu.VMEM((CHUNK, D), jnp.float32),
            pltpu.VMEM((CHUNK,), jnp.float32),
            pltpu.VMEM((ROWS_PT,), jnp.float32),
            pltpu.VMEM_SHARED((NPAD, D), jnp.float32),
            pltpu.VMEM_SHARED((NPAD,), jnp.float32),
        ] + [pltpu.SemaphoreType.DMA] * 12,
    )
    def k(feat_hbm, edges_hbm, sum_out, deg_out,
          srcv, dstv, rowsA, rowsB, rowsC, rowsD, ones_v, zbd_v,
          acc_sh, deg_sh, *sems):
        c = lax.axis_index("c")
        s = lax.axis_index("s")
        wid = c * NS + s
        rows_s = (rowsA, rowsB, rowsC, rowsD)
        isem = sems[0:4]
        gsem = sems[4:8]
        ssem = sems[8:12]

        def zb_body(i, carry):
            r = i // (D // 16)
            col = (i % (D // 16)) * 16
            rowsA[r, pl.ds(col, 16)] = jnp.zeros((16,), jnp.float32)
            return carry
        lax.fori_loop(0, CHUNK * (D // 16), zb_body, 0)

        def zbd_body(i, carry):
            zbd_v[pl.ds(i * 16, 16)] = jnp.zeros((16,), jnp.float32)
            return carry
        lax.fori_loop(0, ROWS_PT // 16, zbd_body, 0)

        def ones_body(i, carry):
            ones_v[pl.ds(i * 16, 16)] = jnp.ones((16,), jnp.float32)
            return carry
        lax.fori_loop(0, CHUNK // 16, ones_body, 0)

        row0_ = pl.multiple_of(s * ROWS_PT, 8)

        def z_body(j, carry):
            r = pl.multiple_of(row0_ + j * CHUNK, 8)
            pltpu.sync_copy(rowsA, acc_sh.at[pl.ds(r, CHUNK), :])
            return carry
        lax.fori_loop(0, ROWS_PT // CHUNK, z_body, 0)
        pltpu.sync_copy(zbd_v, deg_sh.at[pl.ds(row0_, ROWS_PT)])
        plsc.subcore_barrier()

        ebase = wid * EPW

        def start_idx(t, j):
            b = ebase + t * CHUNK
            pltpu.async_copy(edges_hbm.at[pl.ds(b, CHUNK)], srcv.at[j], isem[j])
            pltpu.async_copy(edges_hbm.at[pl.ds(E + b, CHUNK)], dstv.at[j],
                             isem[j])

        def wait_idx(t, j):
            b = ebase + t * CHUNK
            pltpu.make_async_copy(edges_hbm.at[pl.ds(b, CHUNK)], srcv.at[j],
                                  isem[j]).wait()
            pltpu.make_async_copy(edges_hbm.at[pl.ds(E + b, CHUNK)],
                                  dstv.at[j], isem[j]).wait()

        def start_g(j):
            pltpu.async_copy(feat_hbm.at[srcv.at[j]], rows_s[j], gsem[j])

        def wait_g(j):
            pltpu.make_async_copy(feat_hbm.at[srcv.at[j]], rows_s[j],
                                  gsem[j]).wait()

        def start_scat(j):
            pltpu.async_copy(rows_s[j], acc_sh.at[dstv.at[j]], ssem[j],
                             add=True)
            pltpu.async_copy(ones_v, deg_sh.at[dstv.at[j]], ssem[j], add=True)

        def wait_scat(j):
            pltpu.make_async_copy(rows_s[j], acc_sh.at[dstv.at[j]],
                                  ssem[j]).wait()
            pltpu.make_async_copy(ones_v, deg_sh.at[dstv.at[j]],
                                  ssem[j]).wait()

        start_idx(0, 0)
        start_idx(1, 1)
        wait_idx(0, 0)
        start_g(0)

        def e_body(i, carry):
            for j in range(4):
                t = i * 4 + j

                @pl.when(t + 2 < NSTEP)
                def _():
                    @pl.when(t >= 2)
                    def _():
                        wait_scat((j + 2) % 4)
                    start_idx(t + 2, (j + 2) % 4)

                @pl.when(t + 1 < NSTEP)
                def _():
                    wait_idx(t + 1, (j + 1) % 4)
                    start_g((j + 1) % 4)
                wait_g(j)
                start_scat(j)
            return carry
        lax.fori_loop(0, NSTEP // 4, e_body, 0)
        wait_g(0)
        start_scat(0)
        wait_scat(1)
        wait_scat(2)
        wait_scat(3)
        wait_scat(0)
        plsc.subcore_barrier()

        def o_body(j, carry):
            r = pl.multiple_of(row0_ + j * 128, 8)
            pltpu.sync_copy(acc_sh.at[pl.ds(r, 128), :],
                            sum_out.at[c, pl.ds(r, 128), :])
            return carry
        lax.fori_loop(0, ROWS_PT // 128, o_body, 0)
        pltpu.sync_copy(deg_sh.at[pl.ds(row0_, ROWS_PT)],
                        deg_out.at[pl.ds(c * NPAD + row0_, ROWS_PT)])

    return k(feat, edges)


BLK = 256
_HP = lax.Precision.HIGHEST


def _tc_self(feat, W_self, bias):
    def body(f_ref, ws_ref, b_ref, o_ref):
        o_ref[...] = jnp.dot(
            f_ref[...], ws_ref[...], preferred_element_type=jnp.float32,
            precision=_HP) + b_ref[...]

    return pl.pallas_call(
        body,
        grid=((N + BLK - 1) // BLK,),
        in_specs=[
            pl.BlockSpec((BLK, D), lambda i: (i, 0)),
            pl.BlockSpec((D, D), lambda i: (0, 0)),
            pl.BlockSpec((1, D), lambda i: (0, 0)),
        ],
        out_specs=pl.BlockSpec((BLK, D), lambda i: (i, 0)),
        out_shape=jax.ShapeDtypeStruct((N, D), jnp.float32),
    )(feat, W_self, bias)


def _tc_combine(selfp, sum2, deg4, W_neigh):
    def body(p_ref, s_ref, d_ref, wn_ref, o_ref):
        ssum = s_ref[0] + s_ref[1]
        degv = d_ref[0, 0, 0:1, :] + d_ref[1, 0, 0:1, :]
        inv = 1.0 / jnp.maximum(degv, 1.0)
        rr = lax.broadcasted_iota(jnp.int32, (BLK, BLK), 0)
        cc = lax.broadcasted_iota(jnp.int32, (BLK, BLK), 1)
        diag = jnp.where(rr == cc, jnp.broadcast_to(inv, (BLK, BLK)), 0.0)
        h = jnp.dot(diag, ssum, preferred_element_type=jnp.float32,
                    precision=_HP)
        o_ref[...] = p_ref[...] + jnp.dot(
            h, wn_ref[...], preferred_element_type=jnp.float32, precision=_HP)

    return pl.pallas_call(
        body,
        grid=((N + BLK - 1) // BLK,),
        in_specs=[
            pl.BlockSpec((BLK, D), lambda i: (i, 0)),
            pl.BlockSpec((NC, BLK, D), lambda i: (0, i, 0)),
            pl.BlockSpec((NC, 1, 1, BLK), lambda i: (0, i, 0, 0)),
            pl.BlockSpec((D, D), lambda i: (0, 0)),
        ],
        out_specs=pl.BlockSpec((BLK, D), lambda i: (i, 0)),
        out_shape=jax.ShapeDtypeStruct((N, D), jnp.float32),
    )(selfp, sum2, deg4, W_neigh)


def kernel(feat, edge_index, W_self, b_self, W_neigh, b_neigh):
    sum2, deg_flat = _sc_aggregate(feat, edge_index.reshape(-1))
    deg4 = deg_flat.reshape(NC, NPAD // BLK, 1, BLK)
    bias = (b_self + b_neigh).reshape(1, D)
    selfp = _tc_self(feat, W_self, bias)
    return _tc_combine(selfp, sum2, deg4, W_neigh)
